# baseline (speedup 1.0000x reference)
import functools

import jax
import jax.numpy as jnp
from jax import lax
from jax.experimental import pallas as pl
from jax.experimental.pallas import tpu as pltpu
from jax.experimental.pallas import tpu_sc as plsc

B, C, HW = 8, 64, 1024
K = 1024
N = B * HW
COMMITMENT_COST = 0.25
LOSS_SCALE = (1.0 + COMMITMENT_COST) / (N * C)


def _tc_body(x_ref, emb_ref, idx_ref, dmin_ref, loss_ref):
    b = pl.program_id(0)
    xb = x_ref[0]
    emb = emb_ref[...]
    mm = lax.dot_general(emb, xb, (((1,), (0,)), ((), ())),
                         preferred_element_type=jnp.float32)
    x2 = jnp.sum(xb * xb, axis=0)
    e2 = jnp.sum(emb * emb, axis=1)
    d = (x2[None, :] + e2[:, None]) - 2.0 * mm
    dmin = jnp.min(d, axis=0)
    iota_k = lax.broadcasted_iota(jnp.int32, (K, HW), 0)
    idx = jnp.min(jnp.where(d == dmin[None, :], iota_k, K), axis=0)
    idx_ref[0, 0] = idx
    dmin_ref[0, 0] = dmin

    @pl.when(b == 0)
    def _():
        loss_ref[0, 0] = 0.0

    loss_ref[0, 0] += jnp.sum(dmin)

    @pl.when(b == B - 1)
    def _():
        loss_ref[0, 0] *= LOSS_SCALE


def _tc_stage(x3, emb, interpret=False):
    return pl.pallas_call(
        _tc_body,
        grid=(B,),
        in_specs=[
            pl.BlockSpec((1, C, HW), lambda b: (b, 0, 0)),
            pl.BlockSpec((K, C), lambda b: (0, 0)),
        ],
        out_specs=[
            pl.BlockSpec((1, 1, HW), lambda b: (b, 0, 0)),
            pl.BlockSpec((1, 1, HW), lambda b: (b, 0, 0)),
            pl.BlockSpec(memory_space=pltpu.SMEM, block_shape=(1, 1),
                         index_map=lambda b: (0, 0)),
        ],
        out_shape=[
            jax.ShapeDtypeStruct((B, 1, HW), jnp.int32),
            jax.ShapeDtypeStruct((B, 1, HW), jnp.float32),
            jax.ShapeDtypeStruct((1, 1), jnp.float32),
        ],
        interpret=interpret,
    )(x3, emb)


_NC, _NS = 2, 16
_NW = _NC * _NS
_B_PER_W = N // _NW


def _sc_gather(emb, idx, interpret=False):
    mesh = plsc.VectorSubcoreMesh(core_axis_name="c", subcore_axis_name="s")

    @functools.partial(
        pl.kernel,
        mesh=mesh,
        out_type=jax.ShapeDtypeStruct((N, C), jnp.float32),
        scratch_types=[
            pltpu.VMEM((_B_PER_W,), jnp.int32),
            pltpu.VMEM((_B_PER_W, C), jnp.float32),
            pltpu.SemaphoreType.DMA,
        ],
        compiler_params=pltpu.CompilerParams(use_tc_tiling_on_sc=False),
        interpret=interpret,
    )
    def k(emb_hbm, idx_hbm, out_hbm, idx_v, rows_v, sem):
        wid = lax.axis_index("s") * _NC + lax.axis_index("c")
        base = wid * _B_PER_W
        pltpu.sync_copy(idx_hbm.at[pl.ds(base, _B_PER_W)], idx_v)
        pltpu.async_copy(emb_hbm.at[idx_v], rows_v, sem).wait()
        pltpu.sync_copy(rows_v, out_hbm.at[pl.ds(base, _B_PER_W)])

    return k(emb, idx)


def kernel(x, emb):
    x3 = x.reshape(B, C, HW)
    idx3, dmin3, loss = _tc_stage(x3, emb)
    idx_flat = idx3.reshape(N)
    q = _sc_gather(emb, idx_flat)
    quantized = q.reshape(B, 32, 32, C).transpose(0, 3, 1, 2)
    return quantized, loss[0, 0], idx3.reshape(B, HW)


# speedup vs baseline: 1.5559x; 1.5559x over previous
import jax
import jax.numpy as jnp
from jax import lax
from jax.experimental import pallas as pl
from jax.experimental.pallas import tpu as pltpu

B, C, HW = 8, 64, 1024
K = 1024
N = B * HW
COMMITMENT_COST = 0.25
LOSS_SCALE = (1.0 + COMMITMENT_COST) / (N * C)


def _tc_body(x_ref, emb_ref, q_ref, idx_ref, loss_ref):
    b = pl.program_id(0)
    xb = x_ref[0]
    emb = emb_ref[...]
    mm = lax.dot_general(emb, xb, (((1,), (0,)), ((), ())),
                         preferred_element_type=jnp.float32)
    x2 = jnp.sum(xb * xb, axis=0)
    e2 = jnp.sum(emb * emb, axis=1)
    d = (x2[None, :] + e2[:, None]) - 2.0 * mm
    dmin = jnp.min(d, axis=0)
    hit = d == dmin[None, :]
    iota_k = lax.broadcasted_iota(jnp.int32, (K, HW), 0)
    idx = jnp.min(jnp.where(hit, iota_k, K), axis=0)
    idx_ref[0, 0] = idx
    onehot = jnp.where(iota_k == idx[None, :], 1.0, 0.0)
    q_ref[0] = lax.dot_general(emb, onehot, (((0,), (0,)), ((), ())),
                               preferred_element_type=jnp.float32)

    @pl.when(b == 0)
    def _():
        loss_ref[0, 0] = 0.0

    loss_ref[0, 0] += jnp.sum(dmin)

    @pl.when(b == B - 1)
    def _():
        loss_ref[0, 0] *= LOSS_SCALE


def _tc_stage(x3, emb, interpret=False):
    return pl.pallas_call(
        _tc_body,
        grid=(B,),
        in_specs=[
            pl.BlockSpec((1, C, HW), lambda b: (b, 0, 0)),
            pl.BlockSpec((K, C), lambda b: (0, 0)),
        ],
        out_specs=[
            pl.BlockSpec((1, C, HW), lambda b: (b, 0, 0)),
            pl.BlockSpec((1, 1, HW), lambda b: (b, 0, 0)),
            pl.BlockSpec(memory_space=pltpu.SMEM, block_shape=(1, 1),
                         index_map=lambda b: (0, 0)),
        ],
        out_shape=[
            jax.ShapeDtypeStruct((B, C, HW), jnp.float32),
            jax.ShapeDtypeStruct((B, 1, HW), jnp.int32),
            jax.ShapeDtypeStruct((1, 1), jnp.float32),
        ],
        interpret=interpret,
    )(x3, emb)


def kernel(x, emb):
    x3 = x.reshape(B, C, HW)
    q3, idx3, loss = _tc_stage(x3, emb)
    return q3.reshape(B, C, 32, 32), loss[0, 0], idx3.reshape(B, HW)


# speedup vs baseline: 1.7142x; 1.1017x over previous
import jax
import jax.numpy as jnp
from jax import lax
from jax.experimental import pallas as pl
from jax.experimental.pallas import tpu as pltpu

B, C, HW = 8, 64, 1024
K = 1024
N = B * HW
COMMITMENT_COST = 0.25
LOSS_SCALE = (1.0 + COMMITMENT_COST) / (N * C)


def _tournament_argmin(d):
    h = d.shape[0] // 2
    da, db = d[:h], d[h:]
    le = da <= db
    idx = lax.broadcasted_iota(jnp.int32, da.shape, 0) + jnp.where(le, 0, h)
    d = jnp.minimum(da, db)
    while d.shape[0] > 1:
        h = d.shape[0] // 2
        da, db = d[:h], d[h:]
        ia, ib = idx[:h], idx[h:]
        keep_a = (da < db) | ((da == db) & (ia < ib))
        idx = jnp.where(keep_a, ia, ib)
        d = jnp.minimum(da, db)
    return d[0], idx[0]


def _tc_body(x_ref, emb_ref, q_ref, idx_ref, loss_ref):
    b = pl.program_id(0)
    xb = x_ref[0]
    emb = emb_ref[...]
    mm2 = lax.dot_general(emb, xb + xb, (((1,), (0,)), ((), ())),
                          preferred_element_type=jnp.float32)
    x2 = jnp.sum(xb * xb, axis=0)
    e2 = jnp.sum(emb * emb, axis=1)
    d = (x2[None, :] + e2[:, None]) - mm2
    dmin, idx = _tournament_argmin(d)
    idx_ref[pl.ds(b, 1), :] = idx[None, :]
    iota_k = lax.broadcasted_iota(jnp.int32, (K, HW), 0)
    onehot = jnp.where(iota_k == idx[None, :], 1.0, 0.0)
    q_ref[0] = lax.dot_general(emb, onehot, (((0,), (0,)), ((), ())),
                               preferred_element_type=jnp.float32)

    @pl.when(b == 0)
    def _():
        loss_ref[0, 0] = 0.0

    loss_ref[0, 0] += jnp.sum(dmin)

    @pl.when(b == B - 1)
    def _():
        loss_ref[0, 0] *= LOSS_SCALE


def _tc_stage(x3, emb, interpret=False):
    return pl.pallas_call(
        _tc_body,
        grid=(B,),
        in_specs=[
            pl.BlockSpec((1, C, HW), lambda b: (b, 0, 0)),
            pl.BlockSpec((K, C), lambda b: (0, 0)),
        ],
        out_specs=[
            pl.BlockSpec((1, C, HW), lambda b: (b, 0, 0)),
            pl.BlockSpec((B, HW), lambda b: (0, 0)),
            pl.BlockSpec(memory_space=pltpu.SMEM, block_shape=(1, 1),
                         index_map=lambda b: (0, 0)),
        ],
        out_shape=[
            jax.ShapeDtypeStruct((B, C, HW), jnp.float32),
            jax.ShapeDtypeStruct((B, HW), jnp.int32),
            jax.ShapeDtypeStruct((1, 1), jnp.float32),
        ],
        interpret=interpret,
    )(x3, emb)


def kernel(x, emb):
    x3 = x.reshape(B, C, HW)
    q3, idx, loss = _tc_stage(x3, emb)
    return q3.reshape(B, C, 32, 32), loss[0, 0], idx
